# baseline (speedup 1.0000x reference)
import functools

import jax
import jax.numpy as jnp
from jax import lax
from jax.experimental import pallas as pl
from jax.experimental.pallas import tpu as pltpu
from jax.experimental.pallas import tpu_sc as plsc

B = 16384
D = 64
L = 16
_EPS = 1e-12

_info = plsc.get_sparse_core_info()
NC = _info.num_cores
NS = _info.num_subcores
NW = NC * NS
BPW = B // NW
CHUNK = 128
NCHUNK = BPW // CHUNK


def _rsqrt_newton(x):
    i = lax.bitcast_convert_type(x, jnp.int32)
    i = jnp.int32(0x5F3759DF) - lax.shift_right_arithmetic(i, 1)
    y = lax.bitcast_convert_type(i, jnp.float32)
    half_x = x * 0.5
    for _ in range(3):
        y = y * (1.5 - half_x * y * y)
    return y


def _make_sc_kernel():
    mesh = plsc.VectorSubcoreMesh(core_axis_name="c", subcore_axis_name="s")

    @functools.partial(
        pl.kernel,
        mesh=mesh,
        out_type=jax.ShapeDtypeStruct((B,), jnp.float32),
        compiler_params=pltpu.CompilerParams(
            needs_layout_passes=False, use_tc_tiling_on_sc=False),
        scratch_types=[
            pltpu.VMEM((BPW,), jnp.int32),
            pltpu.VMEM((BPW,), jnp.int32),
            pltpu.VMEM((BPW, D), jnp.float32),
            pltpu.VMEM((BPW, D), jnp.float32),
            pltpu.VMEM((BPW,), jnp.float32),
            pltpu.SemaphoreType.DMA,
        ],
    )
    def sc_kernel(fidx_hbm, tidx_hbm, ftab_hbm, ttab_hbm, out_hbm,
                  fidx_v, tidx_v, frows_v, trows_v, out_v, sem):
        wid = lax.axis_index("s") * NC + lax.axis_index("c")
        base = wid * BPW

        pltpu.sync_copy(fidx_hbm.at[pl.ds(base, BPW)], fidx_v)
        pltpu.sync_copy(tidx_hbm.at[pl.ds(base, BPW)], tidx_v)

        copies = []
        for c in range(NCHUNK):
            sl = pl.ds(c * CHUNK, CHUNK)
            copies.append(
                pltpu.async_copy(ftab_hbm.at[fidx_v.at[sl]], frows_v.at[sl], sem))
            copies.append(
                pltpu.async_copy(ttab_hbm.at[tidx_v.at[sl]], trows_v.at[sl], sem))
        for cp in copies:
            cp.wait()

        lane = lax.iota(jnp.int32, L)

        def group_body(g, _):
            acc_dot = jnp.zeros((L,), jnp.float32)
            acc_nf = jnp.zeros((L,), jnp.float32)
            acc_nt = jnp.zeros((L,), jnp.float32)
            for j in range(L):
                r = g * L + j
                f = [frows_v[r, pl.ds(k * L, L)] for k in range(D // L)]
                t = [trows_v[r, pl.ds(k * L, L)] for k in range(D // L)]
                p_dot = f[0] * t[0]
                p_nf = f[0] * f[0]
                p_nt = t[0] * t[0]
                for k in range(1, D // L):
                    p_dot = p_dot + f[k] * t[k]
                    p_nf = p_nf + f[k] * f[k]
                    p_nt = p_nt + t[k] * t[k]
                m = lane == j
                acc_dot = jnp.where(m, jnp.sum(p_dot), acc_dot)
                acc_nf = jnp.where(m, jnp.sum(p_nf), acc_nf)
                acc_nt = jnp.where(m, jnp.sum(p_nt), acc_nt)
            denom = jnp.maximum(acc_nf, _EPS) * jnp.maximum(acc_nt, _EPS)
            out_v[pl.ds(g * L, L)] = acc_dot * _rsqrt_newton(denom)
            return 0

        lax.fori_loop(0, BPW // L, group_body, 0)

        pltpu.sync_copy(out_v, out_hbm.at[pl.ds(base, BPW)])

    return sc_kernel


_sc_kernel = _make_sc_kernel()


@jax.jit
def kernel(file, test, file_table, test_table):
    out = _sc_kernel(file.reshape(B), test.reshape(B), file_table, test_table)
    return out.reshape(B, 1)


# speedup vs baseline: 2.0941x; 2.0941x over previous
import functools

import jax
import jax.numpy as jnp
from jax import lax
from jax.experimental import pallas as pl
from jax.experimental.pallas import tpu as pltpu
from jax.experimental.pallas import tpu_sc as plsc

B = 16384
D = 64
L = 16
_EPS = 1e-12

NUM_FILES = 1000000
NUM_TESTS = 100000

_info = plsc.get_sparse_core_info()
NC = _info.num_cores
NS = _info.num_subcores
NW = NC * NS
BPW = B // NW
CR = 32
NCHUNK = BPW // CR


def _rsqrt_newton(x):
    i = lax.bitcast_convert_type(x, jnp.int32)
    i = jnp.int32(0x5F3759DF) - lax.shift_right_arithmetic(i, 1)
    y = lax.bitcast_convert_type(i, jnp.float32)
    half_x = x * 0.5
    for _ in range(3):
        y = y * (1.5 - half_x * y * y)
    return y


def _make_sc_kernel():
    mesh = plsc.VectorSubcoreMesh(core_axis_name="c", subcore_axis_name="s")

    @functools.partial(
        pl.kernel,
        mesh=mesh,
        out_type=jax.ShapeDtypeStruct((B,), jnp.float32),
        compiler_params=pltpu.CompilerParams(needs_layout_passes=False),
        scratch_types=[
            pltpu.VMEM((BPW,), jnp.int32),
            pltpu.VMEM((BPW,), jnp.int32),
            pltpu.VMEM((BPW,), jnp.int32),
            pltpu.VMEM((BPW,), jnp.int32),
            pltpu.VMEM((CR, 8, D), jnp.float32),
            pltpu.VMEM((CR, 8, D), jnp.float32),
            pltpu.VMEM((BPW,), jnp.float32),
            pltpu.SemaphoreType.DMA,
        ],
    )
    def sc_kernel(fidx_hbm, tidx_hbm, ftab_hbm, ttab_hbm, out_hbm,
                  fidx_v, tidx_v, ftile_v, ttile_v,
                  fbuf_v, tbuf_v, out_v, sem):
        wid = lax.axis_index("s") * NC + lax.axis_index("c")
        base = wid * BPW

        pltpu.sync_copy(fidx_hbm.at[pl.ds(base, BPW)], fidx_v)
        pltpu.sync_copy(tidx_hbm.at[pl.ds(base, BPW)], tidx_v)

        def tile_body(g, _):
            sl = pl.ds(g * L, L)
            ftile_v[sl] = lax.shift_right_logical(fidx_v[sl], 3)
            ttile_v[sl] = lax.shift_right_logical(tidx_v[sl], 3)
            return 0

        lax.fori_loop(0, BPW // L, tile_body, 0)

        lane = lax.iota(jnp.int32, L)

        def chunk_body(c, _):
            cps = []
            for g in range(CR // L):
                gsl = pl.ds(c * CR + g * L, L)
                fvec = ftile_v[gsl]
                tvec = ttile_v[gsl]
                for j in range(L):
                    i = g * L + j
                    cps.append(pltpu.async_copy(
                        ftab_hbm.at[fvec[j]], fbuf_v.at[i], sem))
                    cps.append(pltpu.async_copy(
                        ttab_hbm.at[tvec[j]], tbuf_v.at[i], sem))
            for cp in cps:
                cp.wait()
            for g in range(CR // L):
                gsl = pl.ds(c * CR + g * L, L)
                fvec = fidx_v[gsl]
                tvec = tidx_v[gsl]
                acc_dot = jnp.zeros((L,), jnp.float32)
                acc_nf = jnp.zeros((L,), jnp.float32)
                acc_nt = jnp.zeros((L,), jnp.float32)
                for j in range(L):
                    i = g * L + j
                    fs = fvec[j] & 7
                    ts = tvec[j] & 7
                    f = [fbuf_v[i, fs, pl.ds(k * L, L)] for k in range(D // L)]
                    t = [tbuf_v[i, ts, pl.ds(k * L, L)] for k in range(D // L)]
                    p_dot = f[0] * t[0]
                    p_nf = f[0] * f[0]
                    p_nt = t[0] * t[0]
                    for k in range(1, D // L):
                        p_dot = p_dot + f[k] * t[k]
                        p_nf = p_nf + f[k] * f[k]
                        p_nt = p_nt + t[k] * t[k]
                    m = lane == j
                    acc_dot = jnp.where(m, jnp.sum(p_dot), acc_dot)
                    acc_nf = jnp.where(m, jnp.sum(p_nf), acc_nf)
                    acc_nt = jnp.where(m, jnp.sum(p_nt), acc_nt)
                denom = jnp.maximum(acc_nf, _EPS) * jnp.maximum(acc_nt, _EPS)
                out_v[gsl] = acc_dot * _rsqrt_newton(denom)
            return 0

        lax.fori_loop(0, NCHUNK, chunk_body, 0)

        pltpu.sync_copy(out_v, out_hbm.at[pl.ds(base, BPW)])

    return sc_kernel


_sc_kernel = _make_sc_kernel()


@jax.jit
def kernel(file, test, file_table, test_table):
    ftab3 = file_table.reshape(NUM_FILES // 8, 8, D)
    ttab3 = test_table.reshape(NUM_TESTS // 8, 8, D)
    out = _sc_kernel(file.reshape(B), test.reshape(B), ftab3, ttab3)
    return out.reshape(B, 1)
